# baseline (speedup 1.0000x reference)
import functools
import jax
import jax.numpy as jnp
from jax import lax
from jax.experimental import pallas as pl
from jax.experimental.pallas import tpu as pltpu
from jax.experimental.pallas import tpu_sc as plsc

_CH = 256
_NW = 32


def kernel(x, W, b, masked_value_embedding, pe):
    B, L, _ = x.shape
    E = pe.shape[1]
    bpw = B // _NW
    nch = L // _CH
    x2 = x.reshape(B, L)
    w1 = W.reshape(E)
    mesh = plsc.VectorSubcoreMesh(core_axis_name="c", subcore_axis_name="s")

    @functools.partial(
        pl.kernel,
        mesh=mesh,
        out_type=jax.ShapeDtypeStruct((1, B, L, E), jnp.float32),
        compiler_params=pltpu.CompilerParams(use_tc_tiling_on_sc=False),
        scratch_types=[
            pltpu.VMEM((_CH, E), jnp.float32),
            pltpu.VMEM((_CH, E), jnp.float32),
            pltpu.VMEM((2, _CH, E), jnp.float32),
            pltpu.VMEM((bpw, _CH), jnp.float32),
            pltpu.VMEM((E,), jnp.float32),
            pltpu.VMEM((E,), jnp.float32),
            pltpu.VMEM((E,), jnp.float32),
            pltpu.SemaphoreType.DMA,
        ],
    )
    def sc_fused(x_hbm, w_hbm, b_hbm, mve_hbm, pe_hbm, out_hbm,
                 bpe, mpe, obuf, xbuf, wv, bv, mv, sem):
        wid = lax.axis_index("s") * 2 + lax.axis_index("c")
        pltpu.sync_copy(w_hbm, wv)
        pltpu.sync_copy(b_hbm, bv)
        pltpu.sync_copy(mve_hbm, mv)

        wregs = [wv[pl.ds(16 * j, 16)] for j in range(E // 16)]

        for c in range(nch):
            pltpu.sync_copy(pe_hbm.at[pl.ds(c * _CH, _CH), :], bpe)
            pltpu.sync_copy(pe_hbm.at[pl.ds(c * _CH, _CH), :], mpe)
            pltpu.sync_copy(
                x_hbm.at[pl.ds(wid * bpw, bpw), pl.ds(c * _CH, _CH)], xbuf)

            def add_rows(i, _):
                for j in range(E // 16):
                    sl = pl.ds(16 * j, 16)
                    bpe[i, sl] = bpe[i, sl] + bv[sl]
                    mpe[i, sl] = mpe[i, sl] + mv[sl]
                return 0

            lax.fori_loop(0, _CH, add_rows, 0)

            def one_batch(k, _):
                slot = lax.rem(k, 2)
                bidx = wid * bpw + k

                def one_blk(blk, _2):
                    xv = xbuf[k, pl.ds(16 * blk, 16)]
                    for p in range(16):
                        i = 16 * blk + p
                        v = xv[p]
                        vb = jnp.full((16,), v, jnp.float32)
                        for j in range(E // 16):
                            sl = pl.ds(16 * j, 16)
                            obuf[slot, i, sl] = vb * wregs[j] + bpe[i, sl]

                        @pl.when(v == -10.0)
                        def _mask():
                            for j in range(E // 16):
                                sl = pl.ds(16 * j, 16)
                                obuf[slot, i, sl] = mpe[i, sl]

                    return 0

                lax.fori_loop(0, _CH // 16, one_blk, 0)

                @pl.when(jnp.logical_or(k >= 1, c >= 1))
                def _drain():
                    pltpu.make_async_copy(
                        obuf.at[1 - slot],
                        out_hbm.at[0, bidx, pl.ds(c * _CH, _CH), :],
                        sem).wait()

                pltpu.async_copy(
                    obuf.at[slot],
                    out_hbm.at[0, bidx, pl.ds(c * _CH, _CH), :],
                    sem)
                return 0

            lax.fori_loop(0, bpw, one_batch, 0)

        pltpu.make_async_copy(
            obuf.at[(bpw - 1) % 2],
            out_hbm.at[0, wid * bpw + bpw - 1,
                       pl.ds((nch - 1) * _CH, _CH), :],
            sem).wait()

    return sc_fused(x2, w1, b, masked_value_embedding, pe)


# speedup vs baseline: 1.8796x; 1.8796x over previous
import functools
import jax
import jax.numpy as jnp
from jax import lax
from jax.experimental import pallas as pl
from jax.experimental.pallas import tpu as pltpu
from jax.experimental.pallas import tpu_sc as plsc

_CH = 256
_NW = 32


def kernel(x, W, b, masked_value_embedding, pe):
    B, L, _ = x.shape
    E = pe.shape[1]
    bpw = B // _NW
    nch = L // _CH
    x2 = x.reshape(B, L)
    w1 = W.reshape(E)
    mesh = plsc.VectorSubcoreMesh(core_axis_name="c", subcore_axis_name="s")

    @functools.partial(
        pl.kernel,
        mesh=mesh,
        out_type=jax.ShapeDtypeStruct((1, B, L, E), jnp.float32),
        compiler_params=pltpu.CompilerParams(use_tc_tiling_on_sc=False, needs_layout_passes=False),
        scratch_types=[
            pltpu.VMEM((_CH, E), jnp.float32),
            pltpu.VMEM((_CH, E), jnp.float32),
            pltpu.VMEM((_CH, E), jnp.float32),
            pltpu.VMEM((_CH, E), jnp.float32),
            pltpu.VMEM((bpw, _CH), jnp.float32),
            pltpu.VMEM((E,), jnp.float32),
            pltpu.VMEM((E,), jnp.float32),
            pltpu.VMEM((E,), jnp.float32),
            pltpu.SemaphoreType.DMA,
            pltpu.SemaphoreType.DMA,
        ],
    )
    def sc_fused(x_hbm, w_hbm, b_hbm, mve_hbm, pe_hbm, out_hbm,
                 bpe, mpe, obuf0, obuf1, xbuf, wv, bv, mv, sem0, sem1):
        wid = lax.axis_index("s") * 2 + lax.axis_index("c")
        pltpu.sync_copy(w_hbm, wv)
        pltpu.sync_copy(b_hbm, bv)
        pltpu.sync_copy(mve_hbm, mv)

        wregs = [wv[pl.ds(16 * j, 16)] for j in range(E // 16)]
        idxv = [jnp.full((16,), p, jnp.int32) for p in range(16)]

        def compute_row(k, obuf):
            def one_blk(blk, _2):
                xv = xbuf[k, pl.ds(16 * blk, 16)]
                nmask = plsc.all_reduce_population_count(xv == -10.0)
                has_mask = nmask[0] > 0

                @pl.when(jnp.logical_not(has_mask))
                def _fast():
                    for p in range(16):
                        i = 16 * blk + p
                        vb = xv.at[idxv[p]].get(mode="promise_in_bounds")
                        for j in range(E // 16):
                            sl = pl.ds(16 * j, 16)
                            obuf[i, sl] = vb * wregs[j] + bpe[i, sl]

                @pl.when(has_mask)
                def _slow():
                    fm = jnp.where(xv == -10.0, 1.0, 0.0)
                    for p in range(16):
                        i = 16 * blk + p
                        vb = xv.at[idxv[p]].get(mode="promise_in_bounds")
                        fb = fm.at[idxv[p]].get(mode="promise_in_bounds")
                        for j in range(E // 16):
                            sl = pl.ds(16 * j, 16)
                            t = vb * wregs[j] + bpe[i, sl]
                            obuf[i, sl] = t + fb * (mpe[i, sl] - t)

                return 0

            lax.fori_loop(0, _CH // 16, one_blk, 0)

        for c in range(nch):
            pltpu.sync_copy(pe_hbm.at[pl.ds(c * _CH, _CH), :], bpe)
            pltpu.sync_copy(pe_hbm.at[pl.ds(c * _CH, _CH), :], mpe)
            pltpu.sync_copy(
                x_hbm.at[pl.ds(wid * bpw, bpw), pl.ds(c * _CH, _CH)], xbuf)

            def add_rows(i, _):
                for j in range(E // 16):
                    sl = pl.ds(16 * j, 16)
                    bpe[i, sl] = bpe[i, sl] + bv[sl]
                    mpe[i, sl] = mpe[i, sl] + mv[sl]
                return 0

            lax.fori_loop(0, _CH, add_rows, 0)

            def batch_pair(kk, _):
                k0 = 2 * kk

                def dst(k):
                    return out_hbm.at[0, wid * bpw + k, pl.ds(c * _CH, _CH), :]

                @pl.when((kk >= 1) | (c >= 1))
                def _d0():
                    pltpu.make_async_copy(obuf0, dst(k0), sem0).wait()

                compute_row(k0, obuf0)
                pltpu.async_copy(obuf0, dst(k0), sem0)

                @pl.when((kk >= 1) | (c >= 1))
                def _d1():
                    pltpu.make_async_copy(obuf1, dst(k0 + 1), sem1).wait()

                compute_row(k0 + 1, obuf1)
                pltpu.async_copy(obuf1, dst(k0 + 1), sem1)
                return 0

            lax.fori_loop(0, bpw // 2, batch_pair, 0)

        last = wid * bpw + bpw - 1
        pltpu.make_async_copy(
            obuf0, out_hbm.at[0, last - 1, pl.ds((nch - 1) * _CH, _CH), :],
            sem0).wait()
        pltpu.make_async_copy(
            obuf1, out_hbm.at[0, last, pl.ds((nch - 1) * _CH, _CH), :],
            sem1).wait()

    return sc_fused(x2, w1, b, masked_value_embedding, pe)


# speedup vs baseline: 1.9647x; 1.0453x over previous
import functools
import jax
import jax.numpy as jnp
from jax import lax
from jax.experimental import pallas as pl
from jax.experimental.pallas import tpu as pltpu
from jax.experimental.pallas import tpu_sc as plsc

_CH = 256
_NW = 32


def kernel(x, W, b, masked_value_embedding, pe):
    B, L, _ = x.shape
    E = pe.shape[1]
    bpw = B // _NW
    nch = L // _CH
    x2 = x.reshape(B, L)
    w1 = W.reshape(E)
    mesh = plsc.VectorSubcoreMesh(core_axis_name="c", subcore_axis_name="s")

    @functools.partial(
        pl.kernel,
        mesh=mesh,
        out_type=jax.ShapeDtypeStruct((1, B, L, E), jnp.float32),
        compiler_params=pltpu.CompilerParams(use_tc_tiling_on_sc=False, needs_layout_passes=False),
        scratch_types=[
            pltpu.VMEM((_CH, E), jnp.float32),
            pltpu.VMEM((_CH, E), jnp.float32),
            pltpu.VMEM((_CH, E), jnp.float32),
            pltpu.VMEM((_CH, E), jnp.float32),
            pltpu.VMEM((bpw, _CH), jnp.float32),
            pltpu.VMEM((E,), jnp.float32),
            pltpu.VMEM((E,), jnp.float32),
            pltpu.VMEM((E,), jnp.float32),
            pltpu.SemaphoreType.DMA,
            pltpu.SemaphoreType.DMA,
        ],
    )
    def sc_fused(x_hbm, w_hbm, b_hbm, mve_hbm, pe_hbm, out_hbm,
                 bpe, mpe, obuf0, obuf1, xbuf, wv, bv, mv, sem0, sem1):
        wid = lax.axis_index("s") * 2 + lax.axis_index("c")
        pltpu.sync_copy(w_hbm, wv)
        pltpu.sync_copy(b_hbm, bv)
        pltpu.sync_copy(mve_hbm, mv)

        wregs = [wv[pl.ds(16 * j, 16)] for j in range(E // 16)]
        idxv = [jnp.full((16,), p, jnp.int32) for p in range(16)]

        def compute_row(k, obuf):
            def scan_mask(blk, acc):
                xv = xbuf[k, pl.ds(16 * blk, 16)]
                return acc | jnp.any(xv == -10.0)

            has_mask = lax.fori_loop(0, _CH // 16, scan_mask, False)

            @pl.when(jnp.logical_not(has_mask))
            def _fast():
                def one_blk(blk, _2):
                    xv = xbuf[k, pl.ds(16 * blk, 16)]
                    for p in range(16):
                        i = 16 * blk + p
                        vb = xv.at[idxv[p]].get(mode="promise_in_bounds")
                        for j in range(E // 16):
                            sl = pl.ds(16 * j, 16)
                            obuf[i, sl] = vb * wregs[j] + bpe[i, sl]
                    return 0

                lax.fori_loop(0, _CH // 16, one_blk, 0)

            @pl.when(has_mask)
            def _slow():
                def one_blk(blk, _2):
                    xv = xbuf[k, pl.ds(16 * blk, 16)]
                    fm = jnp.where(xv == -10.0, 1.0, 0.0)
                    for p in range(16):
                        i = 16 * blk + p
                        vb = xv.at[idxv[p]].get(mode="promise_in_bounds")
                        fb = fm.at[idxv[p]].get(mode="promise_in_bounds")
                        for j in range(E // 16):
                            sl = pl.ds(16 * j, 16)
                            t = vb * wregs[j] + bpe[i, sl]
                            obuf[i, sl] = t + fb * (mpe[i, sl] - t)
                    return 0

                lax.fori_loop(0, _CH // 16, one_blk, 0)

        def one_chunk(c, _):
            pltpu.sync_copy(pe_hbm.at[pl.ds(c * _CH, _CH), :], bpe)
            pltpu.sync_copy(pe_hbm.at[pl.ds(c * _CH, _CH), :], mpe)
            pltpu.sync_copy(
                x_hbm.at[pl.ds(wid * bpw, bpw), pl.ds(c * _CH, _CH)], xbuf)

            def add_rows(i, _2):
                for j in range(E // 16):
                    sl = pl.ds(16 * j, 16)
                    bpe[i, sl] = bpe[i, sl] + bv[sl]
                    mpe[i, sl] = mpe[i, sl] + mv[sl]
                return 0

            lax.fori_loop(0, _CH, add_rows, 0)

            def batch_pair(kk, _2):
                k0 = 2 * kk

                def dst(k):
                    return out_hbm.at[0, wid * bpw + k, pl.ds(c * _CH, _CH), :]

                @pl.when((kk >= 1) | (c >= 1))
                def _d0():
                    pltpu.make_async_copy(obuf0, dst(k0), sem0).wait()

                compute_row(k0, obuf0)
                pltpu.async_copy(obuf0, dst(k0), sem0)

                @pl.when((kk >= 1) | (c >= 1))
                def _d1():
                    pltpu.make_async_copy(obuf1, dst(k0 + 1), sem1).wait()

                compute_row(k0 + 1, obuf1)
                pltpu.async_copy(obuf1, dst(k0 + 1), sem1)
                return 0

            lax.fori_loop(0, bpw // 2, batch_pair, 0)
            return 0

        lax.fori_loop(0, nch, one_chunk, 0)

        last = wid * bpw + bpw - 1
        pltpu.make_async_copy(
            obuf0, out_hbm.at[0, last - 1, pl.ds((nch - 1) * _CH, _CH), :],
            sem0).wait()
        pltpu.make_async_copy(
            obuf1, out_hbm.at[0, last, pl.ds((nch - 1) * _CH, _CH), :],
            sem1).wait()

    return sc_fused(x2, w1, b, masked_value_embedding, pe)


# speedup vs baseline: 3.6617x; 1.8637x over previous
import jax
import jax.numpy as jnp
from jax.experimental import pallas as pl
from jax.experimental.pallas import tpu as pltpu

_BB = 8
_BL = 512


def _body(x_ref, w_ref, b_ref, mve_ref, pe_ref, o_ref):
    v3 = x_ref[...][:, :, None]
    w = w_ref[0, :]
    bpe = b_ref[0, :][None, :] + pe_ref[...]
    mpe = mve_ref[0, :][None, :] + pe_ref[...]
    xe = v3 * w[None, None, :] + bpe[None, :, :]
    o_ref[...] = jnp.where(v3 == -10.0, mpe[None, :, :], xe)


def kernel(x, W, b, masked_value_embedding, pe):
    B, L, _ = x.shape
    E = pe.shape[1]
    x2 = x.reshape(B, L)
    w2 = W.reshape(1, E)
    b2 = b.reshape(1, E)
    m2 = masked_value_embedding.reshape(1, E)

    out = pl.pallas_call(
        _body,
        grid=(B // _BB, L // _BL),
        in_specs=[
            pl.BlockSpec((_BB, _BL), lambda i, j: (i, j)),
            pl.BlockSpec((1, E), lambda i, j: (0, 0)),
            pl.BlockSpec((1, E), lambda i, j: (0, 0)),
            pl.BlockSpec((1, E), lambda i, j: (0, 0)),
            pl.BlockSpec((_BL, E), lambda i, j: (j, 0)),
        ],
        out_specs=pl.BlockSpec((_BB, _BL, E), lambda i, j: (i, j, 0)),
        out_shape=jax.ShapeDtypeStruct((B, L, E), jnp.float32),
        compiler_params=pltpu.CompilerParams(
            dimension_semantics=("parallel", "parallel"),
        ),
    )(x2, w2, b2, m2, pe)
    return out[None]


# speedup vs baseline: 4.7658x; 1.3015x over previous
import jax
import jax.numpy as jnp
from jax.experimental import pallas as pl
from jax.experimental.pallas import tpu as pltpu

_BB = 8
_BL = 2048


def _body(x_ref, w_ref, b_ref, mve_ref, pe_ref, o_ref):
    v3 = x_ref[...][:, :, None]
    w = w_ref[0, :]
    bpe = b_ref[0, :][None, :] + pe_ref[...]
    mpe = mve_ref[0, :][None, :] + pe_ref[...]
    xe = v3 * w[None, None, :] + bpe[None, :, :]
    o_ref[...] = jnp.where(v3 == -10.0, mpe[None, :, :], xe)


def kernel(x, W, b, masked_value_embedding, pe):
    B, L, _ = x.shape
    E = pe.shape[1]
    x2 = x.reshape(B, L)
    w2 = W.reshape(1, E)
    b2 = b.reshape(1, E)
    m2 = masked_value_embedding.reshape(1, E)

    out = pl.pallas_call(
        _body,
        grid=(B // _BB, L // _BL),
        in_specs=[
            pl.BlockSpec((_BB, _BL), lambda i, j: (i, j)),
            pl.BlockSpec((1, E), lambda i, j: (0, 0)),
            pl.BlockSpec((1, E), lambda i, j: (0, 0)),
            pl.BlockSpec((1, E), lambda i, j: (0, 0)),
            pl.BlockSpec((_BL, E), lambda i, j: (j, 0)),
        ],
        out_specs=pl.BlockSpec((_BB, _BL, E), lambda i, j: (i, j, 0)),
        out_shape=jax.ShapeDtypeStruct((B, L, E), jnp.float32),
        compiler_params=pltpu.CompilerParams(
            dimension_semantics=("parallel", "parallel"),
        ),
    )(x2, w2, b2, m2, pe)
    return out[None]
